# baseline (speedup 1.0000x reference)
import functools

import jax
import jax.numpy as jnp
from jax import lax
from jax.experimental import pallas as pl
from jax.experimental.pallas import tpu as pltpu
from jax.experimental.pallas import tpu_sc as plsc

N_TOKENS = 32768
NUM_CODES = 512
LATENT_DIM = 384
BETA = 0.25

N_CHUNK_CALLS = 4
CTOK = N_TOKENS // N_CHUNK_CALLS

TILE = 1024
GRID = CTOK // TILE

NC, NS = 2, 16
NW = NC * NS
ROWS_PER_W = CTOK // NW
CHUNK = 128
N_CHUNKS = ROWS_PER_W // CHUNK


def _tc_body(z_ref, w_ref, idx_ref, wn_ref, part_ref, acc_ref, wn_s, wsq_s):
    i = pl.program_id(0)

    @pl.when(i == 0)
    def _():
        w = w_ref[...]
        wn = w / jnp.maximum(
            jnp.sqrt(jnp.sum(w * w, axis=1, keepdims=True)), 1e-12)
        wn_s[...] = wn
        wn_ref[...] = wn
        wsq_s[...] = jnp.sum(wn * wn, axis=1)[None, :]
        acc_ref[0] = 0.0

    wn = wn_s[...]
    z = z_ref[...]
    zn = z / jnp.maximum(jnp.sqrt(jnp.sum(z * z, axis=1, keepdims=True)), 1e-12)
    znsq = jnp.sum(zn * zn, axis=1, keepdims=True)

    s = lax.dot_general(zn, wn, (((1,), (1,)), ((), ())),
                        preferred_element_type=jnp.float32)
    d = (znsq + wsq_s[...]) - 2.0 * s
    idx_ref[0, 0, :] = jnp.argmin(d, axis=1).astype(jnp.int32)

    acc_ref[0] += jnp.sum(jnp.min(d, axis=1))

    @pl.when(i == GRID - 1)
    def _():
        part_ref[...] = jnp.broadcast_to(acc_ref[0], (1, 1))


_tc_call = pl.pallas_call(
    _tc_body,
    grid=(GRID,),
    in_specs=[
        pl.BlockSpec((TILE, LATENT_DIM), lambda i: (i, 0)),
        pl.BlockSpec((NUM_CODES, LATENT_DIM), lambda i: (0, 0)),
    ],
    out_specs=[
        pl.BlockSpec((1, 1, TILE), lambda i: (i, 0, 0)),
        pl.BlockSpec((NUM_CODES, LATENT_DIM), lambda i: (0, 0)),
        pl.BlockSpec((1, 1), lambda i: (0, 0)),
    ],
    out_shape=[
        jax.ShapeDtypeStruct((GRID, 1, TILE), jnp.int32),
        jax.ShapeDtypeStruct((NUM_CODES, LATENT_DIM), jnp.float32),
        jax.ShapeDtypeStruct((1, 1), jnp.float32),
    ],
    scratch_shapes=[
        pltpu.SMEM((1,), jnp.float32),
        pltpu.VMEM((NUM_CODES, LATENT_DIM), jnp.float32),
        pltpu.VMEM((1, NUM_CODES), jnp.float32),
    ],
)


def _make_sc_gather(chunk_id):
    tok_base = chunk_id * CTOK

    @functools.partial(
        pl.kernel,
        mesh=plsc.VectorSubcoreMesh(core_axis_name="c", subcore_axis_name="s"),
        out_type=(),
        scratch_types=[
            pltpu.VMEM((N_CHUNKS, CHUNK), jnp.int32),
            pltpu.VMEM((2, CHUNK, LATENT_DIM), jnp.float32),
            pltpu.SemaphoreType.DMA,
            pltpu.SemaphoreType.DMA,
            pltpu.SemaphoreType.DMA,
        ],
    )
    def _sc_gather(table_hbm, idx_hbm, out_hbm, idx_v, rows_v, sem_g,
                   sem_s0, sem_s1):
        wid = lax.axis_index("s") * NC + lax.axis_index("c")
        pltpu.sync_copy(idx_hbm.at[pl.ds(wid * N_CHUNKS, N_CHUNKS)], idx_v)
        base = tok_base + wid * ROWS_PER_W
        sem_s = (sem_s0, sem_s1)
        scat = [None, None]
        gath = pltpu.async_copy(table_hbm.at[idx_v.at[0]], rows_v.at[0], sem_g)
        for c in range(N_CHUNKS):
            b = c & 1
            gath.wait()
            if c + 1 < N_CHUNKS:
                if scat[1 - b] is not None:
                    scat[1 - b].wait()
                gath = pltpu.async_copy(
                    table_hbm.at[idx_v.at[c + 1]], rows_v.at[1 - b], sem_g)
            scat[b] = pltpu.async_copy(
                rows_v.at[b], out_hbm.at[pl.ds(base + c * CHUNK, CHUNK)],
                sem_s[b])
        scat[0].wait()
        scat[1].wait()

    return _sc_gather


_sc_gathers = [_make_sc_gather(c) for c in range(N_CHUNK_CALLS)]


def kernel(z, weight):
    out_ref = jax.new_ref(jnp.zeros((N_TOKENS, LATENT_DIM), jnp.float32))
    idx_parts = []
    loss_sum = None
    for c in range(N_CHUNK_CALLS):
        z_c = lax.slice_in_dim(z, c * CTOK, (c + 1) * CTOK, axis=0)
        idx3, wn, part = _tc_call(z_c, weight)
        _sc_gathers[c](wn, idx3.reshape(CTOK // CHUNK, CHUNK), out_ref)
        idx_parts.append(idx3.reshape(CTOK))
        loss_sum = part[0, 0] if loss_sum is None else loss_sum + part[0, 0]
    idx = jnp.concatenate(idx_parts)
    loss = loss_sum * ((1.0 + BETA) / (N_TOKENS * LATENT_DIM))
    return (out_ref[...], idx, loss)


# speedup vs baseline: 1.2608x; 1.2608x over previous
import functools

import jax
import jax.numpy as jnp
from jax import lax
from jax.experimental import pallas as pl
from jax.experimental.pallas import tpu as pltpu
from jax.experimental.pallas import tpu_sc as plsc

N_TOKENS = 32768
NUM_CODES = 512
LATENT_DIM = 384
BETA = 0.25

TILE = 1024
GRID = N_TOKENS // TILE

NC, NS = 2, 16
NW = NC * NS
ROWS_PER_W = N_TOKENS // NW
CHUNK = 64
N_CHUNKS = ROWS_PER_W // CHUNK
NBUF = 4


def _tc_body(z_ref, w_ref, idx_ref, wn_ref, loss_ref, acc_ref, wn_s, wsq_s):
    i = pl.program_id(0)

    @pl.when(i == 0)
    def _():
        w = w_ref[...]
        wn = w / jnp.maximum(
            jnp.sqrt(jnp.sum(w * w, axis=1, keepdims=True)), 1e-12)
        wn_s[...] = wn
        wn_ref[...] = wn
        wsq_s[...] = jnp.sum(wn * wn, axis=1)[None, :]
        acc_ref[0] = 0.0

    wn = wn_s[...]
    z = z_ref[...]
    zn = z / jnp.maximum(jnp.sqrt(jnp.sum(z * z, axis=1, keepdims=True)), 1e-12)
    znsq = jnp.sum(zn * zn, axis=1, keepdims=True)

    s = lax.dot_general(zn, wn, (((1,), (1,)), ((), ())),
                        preferred_element_type=jnp.float32)
    d = (znsq + wsq_s[...]) - 2.0 * s

    m = jnp.min(d, axis=1)
    jidx = lax.broadcasted_iota(
        jnp.int32, (TILE, NUM_CODES), 1).astype(jnp.float32)
    cand = jnp.where(d == m[:, None], jidx, float(NUM_CODES))
    idx_ref[0, 0, :] = jnp.min(cand, axis=1).astype(jnp.int32)

    acc_ref[0] += jnp.sum(m)

    @pl.when(i == GRID - 1)
    def _():
        scale = (1.0 + BETA) / (N_TOKENS * LATENT_DIM)
        loss_ref[...] = jnp.broadcast_to(acc_ref[0] * scale, (1, 1))


_tc_call = pl.pallas_call(
    _tc_body,
    grid=(GRID,),
    in_specs=[
        pl.BlockSpec((TILE, LATENT_DIM), lambda i: (i, 0)),
        pl.BlockSpec((NUM_CODES, LATENT_DIM), lambda i: (0, 0)),
    ],
    out_specs=[
        pl.BlockSpec((1, 1, TILE), lambda i: (i, 0, 0)),
        pl.BlockSpec((NUM_CODES, LATENT_DIM), lambda i: (0, 0)),
        pl.BlockSpec((1, 1), lambda i: (0, 0)),
    ],
    out_shape=[
        jax.ShapeDtypeStruct((GRID, 1, TILE), jnp.int32),
        jax.ShapeDtypeStruct((NUM_CODES, LATENT_DIM), jnp.float32),
        jax.ShapeDtypeStruct((1, 1), jnp.float32),
    ],
    scratch_shapes=[
        pltpu.SMEM((1,), jnp.float32),
        pltpu.VMEM((NUM_CODES, LATENT_DIM), jnp.float32),
        pltpu.VMEM((1, NUM_CODES), jnp.float32),
    ],
)


@functools.partial(
    pl.kernel,
    mesh=plsc.VectorSubcoreMesh(core_axis_name="c", subcore_axis_name="s"),
    out_type=jax.ShapeDtypeStruct((N_TOKENS, LATENT_DIM), jnp.float32),
    scratch_types=[
        pltpu.VMEM((N_CHUNKS, CHUNK), jnp.int32),
        pltpu.VMEM((NBUF, CHUNK, LATENT_DIM), jnp.float32),
        pltpu.SemaphoreType.DMA,
        pltpu.SemaphoreType.DMA,
        pltpu.SemaphoreType.DMA,
        pltpu.SemaphoreType.DMA,
        pltpu.SemaphoreType.DMA,
        pltpu.SemaphoreType.DMA,
        pltpu.SemaphoreType.DMA,
        pltpu.SemaphoreType.DMA,
    ],
)
def _sc_gather(table_hbm, idx_hbm, out_hbm, idx_v, rows_v, sem_g0, sem_g1,
               sem_g2, sem_g3, sem_s0, sem_s1, sem_s2, sem_s3):
    wid = lax.axis_index("s") * NC + lax.axis_index("c")
    pltpu.sync_copy(idx_hbm.at[pl.ds(wid * N_CHUNKS, N_CHUNKS)], idx_v)
    base = wid * ROWS_PER_W
    sem_g = (sem_g0, sem_g1, sem_g2, sem_g3)
    sem_s = (sem_s0, sem_s1, sem_s2, sem_s3)
    scat = [None] * NBUF
    gath = [None] * NBUF
    for c in range(2):
        gath[c] = pltpu.async_copy(
            table_hbm.at[idx_v.at[c]], rows_v.at[c], sem_g[c])
    for c in range(N_CHUNKS):
        b = c % NBUF
        gath[b].wait()
        nxt = c + 2
        if nxt < N_CHUNKS:
            nb = nxt % NBUF
            if scat[nb] is not None:
                scat[nb].wait()
            gath[nb] = pltpu.async_copy(
                table_hbm.at[idx_v.at[nxt]], rows_v.at[nb], sem_g[nb])
        scat[b] = pltpu.async_copy(
            rows_v.at[b], out_hbm.at[pl.ds(base + c * CHUNK, CHUNK)],
            sem_s[b])
    for b in range(NBUF):
        if scat[b] is not None:
            scat[b].wait()


def kernel(z, weight):
    idx3, wn, loss = _tc_call(z, weight)
    idx = idx3.reshape(N_TOKENS)
    z_q = _sc_gather(wn, idx3.reshape(N_TOKENS // CHUNK, CHUNK))
    return (z_q, idx, loss[0, 0])
